# baseline (speedup 1.0000x reference)
import functools

import jax
import jax.numpy as jnp
from jax import lax
from jax.experimental import pallas as pl
from jax.experimental.pallas import tpu as pltpu
from jax.experimental.pallas import tpu_sc as plsc

_NC = 2
_NS = 16
_NW = _NC * _NS
_CHUNK = 128
_NBUF = 8


def _init_acc(init, acc, base, rpt):
  off = 0
  while off < rpt:
    sz = min(128, rpt - off)
    pltpu.sync_copy(init.at[pl.ds(0, sz)], acc.at[pl.ds(base + off, sz)])
    off += sz


def _ring_loop(gtab, srcv, dstv, rows, acc, gsem, ssem, cpw):
  for b in range(_NBUF):
    pltpu.async_copy(gtab.at[srcv.at[b]], rows.at[b], gsem[b])

  def round_body(i, carry):
    j0 = i * _NBUF
    for b in range(_NBUF):
      j = j0 + b
      pltpu.make_async_copy(gtab.at[srcv.at[j]], rows.at[b],
                            gsem[b]).wait()
      pltpu.async_copy(rows.at[b], acc.at[dstv.at[j]], ssem[b], add=True)
    for b in range(_NBUF):
      j = j0 + b
      pltpu.make_async_copy(rows.at[b], acc.at[dstv.at[j]],
                            ssem[b]).wait()

      @pl.when(j + _NBUF < cpw)
      def _():
        pltpu.async_copy(gtab.at[srcv.at[j + _NBUF]], rows.at[b],
                         gsem[b])

    return carry

  lax.fori_loop(0, cpw // _NBUF, round_body, 0)


def _edge_pass(npad, d, cpw, gather, ntab=0):
  rpt = npad // _NS
  assert cpw % _NBUF == 0 and (not gather or ntab % _NS == 0)
  tpr = ntab // _NS if gather else 0
  mesh = plsc.VectorSubcoreMesh(core_axis_name="c", subcore_axis_name="s")

  @functools.partial(
      pl.kernel,
      out_type=jax.ShapeDtypeStruct((_NC, npad, d), jnp.float32),
      mesh=mesh,
      compiler_params=pltpu.CompilerParams(use_tc_tiling_on_sc=False),
      scratch_types=[
          pltpu.VMEM((cpw, _CHUNK), jnp.int32),
          pltpu.VMEM((cpw, _CHUNK), jnp.int32),
          pltpu.VMEM((_NBUF, _CHUNK, d), jnp.float32),
          pltpu.VMEM_SHARED((npad, d), jnp.float32),
          pltpu.VMEM_SHARED((max(ntab, _NS), d), jnp.float32),
      ] + [pltpu.SemaphoreType.DMA] * (2 * _NBUF),
  )
  def k(table, srcs, dsts, init, out, srcv, dstv, rows, acc, gtab, *sems):
    gsem = sems[:_NBUF]
    ssem = sems[_NBUF:]
    cid = lax.axis_index("c")
    sid = lax.axis_index("s")
    wid = sid * _NC + cid
    if gather:
      pltpu.sync_copy(srcs.at[wid], srcv)
      pltpu.sync_copy(table.at[pl.ds(sid * tpr, tpr)],
                      gtab.at[pl.ds(sid * tpr, tpr)])
    else:
      pltpu.sync_copy(table.at[pl.ds(0, _CHUNK)], rows.at[0])
    pltpu.sync_copy(dsts.at[wid], dstv)
    _init_acc(init, acc, sid * rpt, rpt)
    plsc.subcore_barrier()

    if gather:
      _ring_loop(gtab, srcv, dstv, rows, acc, gsem, ssem, cpw)
    else:
      def round_body(i, carry):
        j0 = i * _NBUF
        for b in range(_NBUF):
          @pl.when(i > 0)
          def _():
            pltpu.make_async_copy(rows.at[0], acc.at[dstv.at[0]],
                                  ssem[b]).wait()
          pltpu.async_copy(rows.at[0], acc.at[dstv.at[j0 + b]], ssem[b],
                           add=True)
        return carry

      lax.fori_loop(0, cpw // _NBUF, round_body, 0)
      for b in range(_NBUF):
        pltpu.make_async_copy(rows.at[0], acc.at[dstv.at[0]],
                              ssem[b]).wait()

    plsc.subcore_barrier()
    pltpu.sync_copy(acc.at[pl.ds(sid * rpt, rpt)],
                    out.at[cid, pl.ds(sid * rpt, rpt)])

  return k


def _edge_pass_split(npad, dhalf, cpw, ntab):
  rpt = npad // _NS
  assert cpw % _NBUF == 0 and ntab % _NS == 0
  tpr = ntab // _NS
  mesh = plsc.VectorSubcoreMesh(core_axis_name="c", subcore_axis_name="s")

  @functools.partial(
      pl.kernel,
      out_type=jax.ShapeDtypeStruct((_NC, npad, dhalf), jnp.float32),
      mesh=mesh,
      compiler_params=pltpu.CompilerParams(use_tc_tiling_on_sc=False),
      scratch_types=[
          pltpu.VMEM((cpw, _CHUNK), jnp.int32),
          pltpu.VMEM((cpw, _CHUNK), jnp.int32),
          pltpu.VMEM((_NBUF, _CHUNK, dhalf), jnp.float32),
          pltpu.VMEM_SHARED((npad, dhalf), jnp.float32),
          pltpu.VMEM_SHARED((ntab, dhalf), jnp.float32),
      ] + [pltpu.SemaphoreType.DMA] * (2 * _NBUF),
  )
  def k(tab0, tab1, srcs, dsts, init, out, srcv, dstv, rows, acc, gtab,
        *sems):
    gsem = sems[:_NBUF]
    ssem = sems[_NBUF:]
    cid = lax.axis_index("c")
    sid = lax.axis_index("s")
    pltpu.sync_copy(srcs.at[sid], srcv)
    pltpu.sync_copy(dsts.at[sid], dstv)
    sl = pl.ds(sid * tpr, tpr)

    @pl.when(cid == 0)
    def _():
      pltpu.sync_copy(tab0.at[sl], gtab.at[sl])

    @pl.when(cid == 1)
    def _():
      pltpu.sync_copy(tab1.at[sl], gtab.at[sl])

    _init_acc(init, acc, sid * rpt, rpt)
    plsc.subcore_barrier()
    _ring_loop(gtab, srcv, dstv, rows, acc, gsem, ssem, cpw)
    plsc.subcore_barrier()
    pltpu.sync_copy(acc.at[pl.ds(sid * rpt, rpt)],
                    out.at[cid, pl.ds(sid * rpt, rpt)])

  return k


def _ka1_body(x_ref, w_ref, h_ref):
  h_ref[...] = jnp.dot(x_ref[...], w_ref[...],
                       preferred_element_type=jnp.float32)


def _ka2_body(h_ref, dp_ref, ga_ref, gb_ref, di_ref):
  deg = dp_ref[0, :, 0:1] + dp_ref[1, :, 0:1] + 1.0
  dinv = lax.rsqrt(deg)
  di_ref[...] = dinv
  g = h_ref[...] * dinv
  half = g.shape[1] // 2
  ga_ref[...] = g[:, :half]
  gb_ref[...] = g[:, half:]


def _kb_body(p_ref, ga_ref, gb_ref, di_ref, b1_ref, w2_ref, s_ref, g2_ref):
  di = di_ref[...]
  conv = jnp.concatenate([p_ref[0] + ga_ref[...], p_ref[1] + gb_ref[...]],
                         axis=1)
  pre = di * conv + b1_ref[...]
  h2 = jnp.maximum(pre, 0.0) * s_ref[...]
  g2_ref[...] = di * jnp.dot(h2, w2_ref[...],
                             preferred_element_type=jnp.float32)


def _kc_body(p_ref, g2_ref, di_ref, b2_ref, o_ref):
  o = di_ref[...] * (p_ref[0] + p_ref[1] + g2_ref[...]) + b2_ref[...]
  m = jnp.max(o, axis=1, keepdims=True)
  lse = jnp.log(jnp.sum(jnp.exp(o - m), axis=1, keepdims=True)) + m
  o_ref[...] = o - lse


def kernel(x, edge_index, idx, W1, b1, W2, b2):
  n, d_in = x.shape
  dh = W1.shape[1]
  dc = W2.shape[1]
  e = edge_index.shape[1]

  npad = 128 * (-(-(n + 1) // 128))
  cpw = _NBUF * (-(-e // (_NW * _CHUNK * _NBUF)))
  epad = _NW * cpw * _CHUNK
  cps = _NC * cpw

  src = jnp.concatenate(
      [edge_index[0], jnp.zeros((epad - e,), jnp.int32)])
  dst = jnp.concatenate(
      [edge_index[1], jnp.full((epad - e,), n, jnp.int32)])
  src32 = src.reshape(_NW, cpw, _CHUNK)
  dst32 = dst.reshape(_NW, cpw, _CHUNK)
  src16 = src.reshape(_NS, cps, _CHUNK)
  dst16 = dst.reshape(_NS, cps, _CHUNK)

  ones8 = jnp.ones((_CHUNK, 8), jnp.float32)
  zeros8 = jnp.zeros((_CHUNK, 8), jnp.float32)
  degp = _edge_pass(npad, 8, cpw, gather=False)(ones8, dst32, dst32, zeros8)

  bm = 2000
  grid = (n // bm,)
  half = dh // 2
  h1 = pl.pallas_call(
      _ka1_body,
      grid=grid,
      in_specs=[
          pl.BlockSpec((bm, d_in), lambda i: (i, 0)),
          pl.BlockSpec((d_in, dh), lambda i: (0, 0)),
      ],
      out_specs=pl.BlockSpec((bm, dh), lambda i: (i, 0)),
      out_shape=jax.ShapeDtypeStruct((n, dh), jnp.float32),
  )(x, W1)
  g1a, g1b, dinv = pl.pallas_call(
      _ka2_body,
      grid=grid,
      in_specs=[
          pl.BlockSpec((bm, dh), lambda i: (i, 0)),
          pl.BlockSpec((2, bm, 8), lambda i: (0, i, 0)),
      ],
      out_specs=[
          pl.BlockSpec((bm, half), lambda i: (i, 0)),
          pl.BlockSpec((bm, half), lambda i: (i, 0)),
          pl.BlockSpec((bm, 1), lambda i: (i, 0)),
      ],
      out_shape=[
          jax.ShapeDtypeStruct((n, half), jnp.float32),
          jax.ShapeDtypeStruct((n, half), jnp.float32),
          jax.ShapeDtypeStruct((n, 1), jnp.float32),
      ],
  )(h1, degp)

  acc1 = _edge_pass_split(npad, half, cps, ntab=n)(
      g1a, g1b, src16, dst16, jnp.zeros((_CHUNK, half), jnp.float32))

  s = jnp.where(idx == 0, 1.0, -1.0).astype(jnp.float32).reshape(1, 1)
  g2 = pl.pallas_call(
      _kb_body,
      grid=grid,
      in_specs=[
          pl.BlockSpec((2, bm, half), lambda i: (0, i, 0)),
          pl.BlockSpec((bm, half), lambda i: (i, 0)),
          pl.BlockSpec((bm, half), lambda i: (i, 0)),
          pl.BlockSpec((bm, 1), lambda i: (i, 0)),
          pl.BlockSpec((1, dh), lambda i: (0, 0)),
          pl.BlockSpec((dh, dc), lambda i: (0, 0)),
          pl.BlockSpec((1, 1), lambda i: (0, 0)),
      ],
      out_specs=pl.BlockSpec((bm, dc), lambda i: (i, 0)),
      out_shape=jax.ShapeDtypeStruct((n, dc), jnp.float32),
  )(acc1, g1a, g1b, dinv, b1.reshape(1, dh), W2, s)

  acc2 = _edge_pass(npad, dc, cpw, gather=True, ntab=n)(
      g2, src32, dst32, jnp.zeros((_CHUNK, dc), jnp.float32))

  out = pl.pallas_call(
      _kc_body,
      grid=grid,
      in_specs=[
          pl.BlockSpec((2, bm, dc), lambda i: (0, i, 0)),
          pl.BlockSpec((bm, dc), lambda i: (i, 0)),
          pl.BlockSpec((bm, 1), lambda i: (i, 0)),
          pl.BlockSpec((1, dc), lambda i: (0, 0)),
      ],
      out_specs=pl.BlockSpec((bm, dc), lambda i: (i, 0)),
      out_shape=jax.ShapeDtypeStruct((n, dc), jnp.float32),
  )(acc2, g2, dinv, b2.reshape(1, dc))

  return out


# speedup vs baseline: 1.0070x; 1.0070x over previous
import functools

import jax
import jax.numpy as jnp
from jax import lax
from jax.experimental import pallas as pl
from jax.experimental.pallas import tpu as pltpu
from jax.experimental.pallas import tpu_sc as plsc

_NC = 2
_NS = 16
_NW = _NC * _NS
_CHUNK = 128
_NBUF = 8


def _init_acc(init, acc, base, rpt):
  off = 0
  while off < rpt:
    sz = min(128, rpt - off)
    pltpu.sync_copy(init.at[pl.ds(0, sz)], acc.at[pl.ds(base + off, sz)])
    off += sz


def _ring_loop(gtab, srcv, dstv, rows, acc, gsem, ssem, cpw):
  for b in range(_NBUF):
    pltpu.async_copy(gtab.at[srcv.at[b]], rows.at[b], gsem[b])

  def round_body(i, carry):
    j0 = i * _NBUF
    for b in range(_NBUF):
      j = j0 + b
      pltpu.make_async_copy(gtab.at[srcv.at[j]], rows.at[b],
                            gsem[b]).wait()
      pltpu.async_copy(rows.at[b], acc.at[dstv.at[j]], ssem[b], add=True)
    for b in range(_NBUF):
      j = j0 + b
      pltpu.make_async_copy(rows.at[b], acc.at[dstv.at[j]],
                            ssem[b]).wait()

      @pl.when(j + _NBUF < cpw)
      def _():
        pltpu.async_copy(gtab.at[srcv.at[j + _NBUF]], rows.at[b],
                         gsem[b])

    return carry

  lax.fori_loop(0, cpw // _NBUF, round_body, 0)


def _edge_pass(npad, d, cpw, gather, ntab=0):
  rpt = npad // _NS
  assert cpw % _NBUF == 0 and (not gather or ntab % _NS == 0)
  tpr = ntab // _NS if gather else 0
  mesh = plsc.VectorSubcoreMesh(core_axis_name="c", subcore_axis_name="s")

  @functools.partial(
      pl.kernel,
      out_type=jax.ShapeDtypeStruct((_NC, npad, d), jnp.float32),
      mesh=mesh,
      compiler_params=pltpu.CompilerParams(use_tc_tiling_on_sc=False),
      scratch_types=[
          pltpu.VMEM((cpw, _CHUNK), jnp.int32),
          pltpu.VMEM((cpw, _CHUNK), jnp.int32),
          pltpu.VMEM((_NBUF, _CHUNK, d), jnp.float32),
          pltpu.VMEM_SHARED((npad, d), jnp.float32),
          pltpu.VMEM_SHARED((max(ntab, _NS), d), jnp.float32),
      ] + [pltpu.SemaphoreType.DMA] * (2 * _NBUF),
  )
  def k(table, srcs, dsts, init, out, srcv, dstv, rows, acc, gtab, *sems):
    gsem = sems[:_NBUF]
    ssem = sems[_NBUF:]
    cid = lax.axis_index("c")
    sid = lax.axis_index("s")
    wid = sid * _NC + cid
    if gather:
      pltpu.sync_copy(srcs.at[wid], srcv)
      pltpu.sync_copy(table.at[pl.ds(sid * tpr, tpr)],
                      gtab.at[pl.ds(sid * tpr, tpr)])
    else:
      pltpu.sync_copy(table.at[pl.ds(0, _CHUNK)], rows.at[0])
    pltpu.sync_copy(dsts.at[wid], dstv)
    _init_acc(init, acc, sid * rpt, rpt)
    plsc.subcore_barrier()

    if gather:
      _ring_loop(gtab, srcv, dstv, rows, acc, gsem, ssem, cpw)
    else:
      def round_body(i, carry):
        j0 = i * _NBUF
        for b in range(_NBUF):
          @pl.when(i > 0)
          def _():
            pltpu.make_async_copy(rows.at[0], acc.at[dstv.at[0]],
                                  ssem[b]).wait()
          pltpu.async_copy(rows.at[0], acc.at[dstv.at[j0 + b]], ssem[b],
                           add=True)
        return carry

      lax.fori_loop(0, cpw // _NBUF, round_body, 0)
      for b in range(_NBUF):
        pltpu.make_async_copy(rows.at[0], acc.at[dstv.at[0]],
                              ssem[b]).wait()

    plsc.subcore_barrier()
    pltpu.sync_copy(acc.at[pl.ds(sid * rpt, rpt)],
                    out.at[cid, pl.ds(sid * rpt, rpt)])

  return k


def _edge_pass_split(npad, dhalf, cpw, ntab):
  rpt = npad // _NS
  assert cpw % _NBUF == 0 and ntab % _NS == 0
  tpr = ntab // _NS
  mesh = plsc.VectorSubcoreMesh(core_axis_name="c", subcore_axis_name="s")

  @functools.partial(
      pl.kernel,
      out_type=jax.ShapeDtypeStruct((_NC, npad, dhalf), jnp.float32),
      mesh=mesh,
      compiler_params=pltpu.CompilerParams(use_tc_tiling_on_sc=False),
      scratch_types=[
          pltpu.VMEM((cpw, _CHUNK), jnp.int32),
          pltpu.VMEM((cpw, _CHUNK), jnp.int32),
          pltpu.VMEM((_NBUF, _CHUNK, dhalf), jnp.float32),
          pltpu.VMEM_SHARED((npad, dhalf), jnp.float32),
          pltpu.VMEM_SHARED((ntab, dhalf), jnp.float32),
      ] + [pltpu.SemaphoreType.DMA] * (2 * _NBUF),
  )
  def k(tab0, tab1, srcs, dsts, init, out, srcv, dstv, rows, acc, gtab,
        *sems):
    gsem = sems[:_NBUF]
    ssem = sems[_NBUF:]
    cid = lax.axis_index("c")
    sid = lax.axis_index("s")
    pltpu.sync_copy(srcs.at[sid], srcv)
    pltpu.sync_copy(dsts.at[sid], dstv)
    sl = pl.ds(sid * tpr, tpr)

    @pl.when(cid == 0)
    def _():
      pltpu.sync_copy(tab0.at[sl], gtab.at[sl])

    @pl.when(cid == 1)
    def _():
      pltpu.sync_copy(tab1.at[sl], gtab.at[sl])

    _init_acc(init, acc, sid * rpt, rpt)
    plsc.subcore_barrier()
    _ring_loop(gtab, srcv, dstv, rows, acc, gsem, ssem, cpw)
    plsc.subcore_barrier()
    pltpu.sync_copy(acc.at[pl.ds(sid * rpt, rpt)],
                    out.at[cid, pl.ds(sid * rpt, rpt)])

  return k


def _ka_body(x_ref, w_ref, dp_ref, ga_ref, gb_ref, di_ref):
  deg = dp_ref[0, :, 0:1] + dp_ref[1, :, 0:1] + 1.0
  dinv = lax.rsqrt(deg)
  di_ref[...] = dinv
  h = jnp.dot(x_ref[...], w_ref[...], preferred_element_type=jnp.float32)
  g = h * dinv
  half = g.shape[1] // 2
  ga_ref[...] = g[:, :half]
  gb_ref[...] = g[:, half:]


def _kb_body(p_ref, ga_ref, gb_ref, di_ref, b1_ref, w2_ref, s_ref, g2_ref):
  di = di_ref[...]
  conv = jnp.concatenate([p_ref[0] + ga_ref[...], p_ref[1] + gb_ref[...]],
                         axis=1)
  pre = di * conv + b1_ref[...]
  h2 = jnp.maximum(pre, 0.0) * s_ref[...]
  g2_ref[...] = di * jnp.dot(h2, w2_ref[...],
                             preferred_element_type=jnp.float32)


def _kc_body(p_ref, g2_ref, di_ref, b2_ref, o_ref):
  o = di_ref[...] * (p_ref[0] + p_ref[1] + g2_ref[...]) + b2_ref[...]
  m = jnp.max(o, axis=1, keepdims=True)
  lse = jnp.log(jnp.sum(jnp.exp(o - m), axis=1, keepdims=True)) + m
  o_ref[...] = o - lse


def kernel(x, edge_index, idx, W1, b1, W2, b2):
  n, d_in = x.shape
  dh = W1.shape[1]
  dc = W2.shape[1]
  e = edge_index.shape[1]

  npad = 128 * (-(-(n + 1) // 128))
  cpw = _NBUF * (-(-e // (_NW * _CHUNK * _NBUF)))
  epad = _NW * cpw * _CHUNK
  cps = _NC * cpw

  src = jnp.concatenate(
      [edge_index[0], jnp.zeros((epad - e,), jnp.int32)])
  dst = jnp.concatenate(
      [edge_index[1], jnp.full((epad - e,), n, jnp.int32)])
  src32 = src.reshape(_NW, cpw, _CHUNK)
  dst32 = dst.reshape(_NW, cpw, _CHUNK)
  src16 = src.reshape(_NS, cps, _CHUNK)
  dst16 = dst.reshape(_NS, cps, _CHUNK)

  ones8 = jnp.ones((_CHUNK, 8), jnp.float32)
  zeros8 = jnp.zeros((_CHUNK, 8), jnp.float32)
  degp = _edge_pass(npad, 8, cpw, gather=False)(ones8, dst32, dst32, zeros8)

  bm = 2000
  grid = (n // bm,)
  half = dh // 2
  g1a, g1b, dinv = pl.pallas_call(
      _ka_body,
      grid=grid,
      in_specs=[
          pl.BlockSpec((bm, d_in), lambda i: (i, 0)),
          pl.BlockSpec((d_in, dh), lambda i: (0, 0)),
          pl.BlockSpec((2, bm, 8), lambda i: (0, i, 0)),
      ],
      out_specs=[
          pl.BlockSpec((bm, half), lambda i: (i, 0)),
          pl.BlockSpec((bm, half), lambda i: (i, 0)),
          pl.BlockSpec((bm, 1), lambda i: (i, 0)),
      ],
      out_shape=[
          jax.ShapeDtypeStruct((n, half), jnp.float32),
          jax.ShapeDtypeStruct((n, half), jnp.float32),
          jax.ShapeDtypeStruct((n, 1), jnp.float32),
      ],
  )(x, W1, degp)

  acc1 = _edge_pass_split(npad, half, cps, ntab=n)(
      g1a, g1b, src16, dst16, jnp.zeros((_CHUNK, half), jnp.float32))

  s = jnp.where(idx == 0, 1.0, -1.0).astype(jnp.float32).reshape(1, 1)
  g2 = pl.pallas_call(
      _kb_body,
      grid=grid,
      in_specs=[
          pl.BlockSpec((2, bm, half), lambda i: (0, i, 0)),
          pl.BlockSpec((bm, half), lambda i: (i, 0)),
          pl.BlockSpec((bm, half), lambda i: (i, 0)),
          pl.BlockSpec((bm, 1), lambda i: (i, 0)),
          pl.BlockSpec((1, dh), lambda i: (0, 0)),
          pl.BlockSpec((dh, dc), lambda i: (0, 0)),
          pl.BlockSpec((1, 1), lambda i: (0, 0)),
      ],
      out_specs=pl.BlockSpec((bm, dc), lambda i: (i, 0)),
      out_shape=jax.ShapeDtypeStruct((n, dc), jnp.float32),
  )(acc1, g1a, g1b, dinv, b1.reshape(1, dh), W2, s)

  acc2 = _edge_pass(npad, dc, cpw, gather=True, ntab=n)(
      g2, src32, dst32, jnp.zeros((_CHUNK, dc), jnp.float32))

  out = pl.pallas_call(
      _kc_body,
      grid=grid,
      in_specs=[
          pl.BlockSpec((2, bm, dc), lambda i: (0, i, 0)),
          pl.BlockSpec((bm, dc), lambda i: (i, 0)),
          pl.BlockSpec((bm, 1), lambda i: (i, 0)),
          pl.BlockSpec((1, dc), lambda i: (0, 0)),
      ],
      out_specs=pl.BlockSpec((bm, dc), lambda i: (i, 0)),
      out_shape=jax.ShapeDtypeStruct((n, dc), jnp.float32),
  )(acc2, g2, dinv, b2.reshape(1, dc))

  return out


# speedup vs baseline: 1.0413x; 1.0341x over previous
import functools

import jax
import jax.numpy as jnp
from jax import lax
from jax.experimental import pallas as pl
from jax.experimental.pallas import tpu as pltpu
from jax.experimental.pallas import tpu_sc as plsc

_NC = 2
_NS = 16
_NW = _NC * _NS
_CHUNK = 125
_NBUF = 8


def _init_acc(init, acc, base, rpt):
  off = 0
  while off < rpt:
    sz = min(128, rpt - off)
    pltpu.sync_copy(init.at[pl.ds(0, sz)], acc.at[pl.ds(base + off, sz)])
    off += sz


def _ring_loop(gtab, srcv, dstv, rows, acc, gsem, ssem, cpw):
  for b in range(_NBUF):
    pltpu.async_copy(gtab.at[srcv.at[b]], rows.at[b], gsem[b])

  def round_body(i, carry):
    j0 = i * _NBUF
    for b in range(_NBUF):
      j = j0 + b
      pltpu.make_async_copy(gtab.at[srcv.at[j]], rows.at[b],
                            gsem[b]).wait()
      pltpu.async_copy(rows.at[b], acc.at[dstv.at[j]], ssem[b], add=True)
    for b in range(_NBUF):
      j = j0 + b
      pltpu.make_async_copy(rows.at[b], acc.at[dstv.at[j]],
                            ssem[b]).wait()

      @pl.when(j + _NBUF < cpw)
      def _():
        pltpu.async_copy(gtab.at[srcv.at[j + _NBUF]], rows.at[b],
                         gsem[b])

    return carry

  lax.fori_loop(0, cpw // _NBUF, round_body, 0)


def _edge_pass(npad, d, cpw, gather, ntab=0):
  rpt = npad // _NS
  assert cpw % _NBUF == 0 and (not gather or ntab % _NS == 0)
  tpr = ntab // _NS if gather else 0
  mesh = plsc.VectorSubcoreMesh(core_axis_name="c", subcore_axis_name="s")

  @functools.partial(
      pl.kernel,
      out_type=jax.ShapeDtypeStruct((_NC, npad, d), jnp.float32),
      mesh=mesh,
      compiler_params=pltpu.CompilerParams(use_tc_tiling_on_sc=False),
      scratch_types=[
          pltpu.VMEM((cpw, _CHUNK), jnp.int32),
          pltpu.VMEM((cpw, _CHUNK), jnp.int32),
          pltpu.VMEM((_NBUF, _CHUNK, d), jnp.float32),
          pltpu.VMEM_SHARED((npad, d), jnp.float32),
          pltpu.VMEM_SHARED((max(ntab, _NS), d), jnp.float32),
      ] + [pltpu.SemaphoreType.DMA] * (2 * _NBUF),
  )
  def k(table, srcs, dsts, init, out, srcv, dstv, rows, acc, gtab, *sems):
    gsem = sems[:_NBUF]
    ssem = sems[_NBUF:]
    cid = lax.axis_index("c")
    sid = lax.axis_index("s")
    wid = sid * _NC + cid
    if gather:
      pltpu.sync_copy(srcs.at[wid], srcv)
      pltpu.sync_copy(table.at[pl.ds(sid * tpr, tpr)],
                      gtab.at[pl.ds(sid * tpr, tpr)])
    else:
      pltpu.sync_copy(table.at[pl.ds(0, _CHUNK)], rows.at[0])
    pltpu.sync_copy(dsts.at[wid], dstv)
    _init_acc(init, acc, sid * rpt, rpt)
    plsc.subcore_barrier()

    if gather:
      _ring_loop(gtab, srcv, dstv, rows, acc, gsem, ssem, cpw)
    else:
      def round_body(i, carry):
        j0 = i * _NBUF
        for b in range(_NBUF):
          @pl.when(i > 0)
          def _():
            pltpu.make_async_copy(rows.at[0], acc.at[dstv.at[0]],
                                  ssem[b]).wait()
          pltpu.async_copy(rows.at[0], acc.at[dstv.at[j0 + b]], ssem[b],
                           add=True)
        return carry

      lax.fori_loop(0, cpw // _NBUF, round_body, 0)
      for b in range(_NBUF):
        pltpu.make_async_copy(rows.at[0], acc.at[dstv.at[0]],
                              ssem[b]).wait()

    plsc.subcore_barrier()
    pltpu.sync_copy(acc.at[pl.ds(sid * rpt, rpt)],
                    out.at[cid, pl.ds(sid * rpt, rpt)])

  return k


def _edge_pass_split(npad, dhalf, cpw, ntab):
  rpt = npad // _NS
  assert cpw % _NBUF == 0 and ntab % _NS == 0
  tpr = ntab // _NS
  mesh = plsc.VectorSubcoreMesh(core_axis_name="c", subcore_axis_name="s")

  @functools.partial(
      pl.kernel,
      out_type=jax.ShapeDtypeStruct((_NC, npad, dhalf), jnp.float32),
      mesh=mesh,
      compiler_params=pltpu.CompilerParams(use_tc_tiling_on_sc=False),
      scratch_types=[
          pltpu.VMEM((cpw, _CHUNK), jnp.int32),
          pltpu.VMEM((cpw, _CHUNK), jnp.int32),
          pltpu.VMEM((_NBUF, _CHUNK, dhalf), jnp.float32),
          pltpu.VMEM_SHARED((npad, dhalf), jnp.float32),
          pltpu.VMEM_SHARED((ntab, dhalf), jnp.float32),
      ] + [pltpu.SemaphoreType.DMA] * (2 * _NBUF),
  )
  def k(tab0, tab1, srcs, dsts, init, out, srcv, dstv, rows, acc, gtab,
        *sems):
    gsem = sems[:_NBUF]
    ssem = sems[_NBUF:]
    cid = lax.axis_index("c")
    sid = lax.axis_index("s")
    pltpu.sync_copy(srcs.at[sid], srcv)
    pltpu.sync_copy(dsts.at[sid], dstv)
    sl = pl.ds(sid * tpr, tpr)

    @pl.when(cid == 0)
    def _():
      pltpu.sync_copy(tab0.at[sl], gtab.at[sl])

    @pl.when(cid == 1)
    def _():
      pltpu.sync_copy(tab1.at[sl], gtab.at[sl])

    _init_acc(init, acc, sid * rpt, rpt)
    plsc.subcore_barrier()
    _ring_loop(gtab, srcv, dstv, rows, acc, gsem, ssem, cpw)
    plsc.subcore_barrier()
    pltpu.sync_copy(acc.at[pl.ds(sid * rpt, rpt)],
                    out.at[cid, pl.ds(sid * rpt, rpt)])

  return k


def _ka_body(x_ref, w_ref, dp_ref, ga_ref, gb_ref, di_ref):
  deg = dp_ref[0, :, 0:1] + dp_ref[1, :, 0:1] + 1.0
  dinv = lax.rsqrt(deg)
  di_ref[...] = dinv
  h = jnp.dot(x_ref[...], w_ref[...], preferred_element_type=jnp.float32)
  g = h * dinv
  half = g.shape[1] // 2
  ga_ref[...] = g[:, :half]
  gb_ref[...] = g[:, half:]


def _kb_body(p_ref, ga_ref, gb_ref, di_ref, b1_ref, w2_ref, s_ref, g2_ref):
  di = di_ref[...]
  conv = jnp.concatenate([p_ref[0] + ga_ref[...], p_ref[1] + gb_ref[...]],
                         axis=1)
  pre = di * conv + b1_ref[...]
  h2 = jnp.maximum(pre, 0.0) * s_ref[...]
  g2_ref[...] = di * jnp.dot(h2, w2_ref[...],
                             preferred_element_type=jnp.float32)


def _kc_body(p_ref, g2_ref, di_ref, b2_ref, o_ref):
  o = di_ref[...] * (p_ref[0] + p_ref[1] + g2_ref[...]) + b2_ref[...]
  m = jnp.max(o, axis=1, keepdims=True)
  lse = jnp.log(jnp.sum(jnp.exp(o - m), axis=1, keepdims=True)) + m
  o_ref[...] = o - lse


def kernel(x, edge_index, idx, W1, b1, W2, b2):
  n, d_in = x.shape
  dh = W1.shape[1]
  dc = W2.shape[1]
  e = edge_index.shape[1]

  npad = n
  assert e % (_NW * _CHUNK) == 0
  cpw = e // (_NW * _CHUNK)
  cps = _NC * cpw

  src32 = edge_index[0].reshape(_NW, cpw, _CHUNK)
  dst32 = edge_index[1].reshape(_NW, cpw, _CHUNK)
  src16 = edge_index[0].reshape(_NS, cps, _CHUNK)
  dst16 = edge_index[1].reshape(_NS, cps, _CHUNK)

  ones8 = jnp.ones((_CHUNK, 8), jnp.float32)
  zeros8 = jnp.zeros((_CHUNK, 8), jnp.float32)
  degp = _edge_pass(npad, 8, cpw, gather=False)(ones8, dst32, dst32, zeros8)

  bm = 2000
  grid = (n // bm,)
  half = dh // 2
  g1a, g1b, dinv = pl.pallas_call(
      _ka_body,
      grid=grid,
      in_specs=[
          pl.BlockSpec((bm, d_in), lambda i: (i, 0)),
          pl.BlockSpec((d_in, dh), lambda i: (0, 0)),
          pl.BlockSpec((2, bm, 8), lambda i: (0, i, 0)),
      ],
      out_specs=[
          pl.BlockSpec((bm, half), lambda i: (i, 0)),
          pl.BlockSpec((bm, half), lambda i: (i, 0)),
          pl.BlockSpec((bm, 1), lambda i: (i, 0)),
      ],
      out_shape=[
          jax.ShapeDtypeStruct((n, half), jnp.float32),
          jax.ShapeDtypeStruct((n, half), jnp.float32),
          jax.ShapeDtypeStruct((n, 1), jnp.float32),
      ],
  )(x, W1, degp)

  acc1 = _edge_pass_split(npad, half, cps, ntab=n)(
      g1a, g1b, src16, dst16, jnp.zeros((_CHUNK, half), jnp.float32))

  s = jnp.where(idx == 0, 1.0, -1.0).astype(jnp.float32).reshape(1, 1)
  g2 = pl.pallas_call(
      _kb_body,
      grid=grid,
      in_specs=[
          pl.BlockSpec((2, bm, half), lambda i: (0, i, 0)),
          pl.BlockSpec((bm, half), lambda i: (i, 0)),
          pl.BlockSpec((bm, half), lambda i: (i, 0)),
          pl.BlockSpec((bm, 1), lambda i: (i, 0)),
          pl.BlockSpec((1, dh), lambda i: (0, 0)),
          pl.BlockSpec((dh, dc), lambda i: (0, 0)),
          pl.BlockSpec((1, 1), lambda i: (0, 0)),
      ],
      out_specs=pl.BlockSpec((bm, dc), lambda i: (i, 0)),
      out_shape=jax.ShapeDtypeStruct((n, dc), jnp.float32),
  )(acc1, g1a, g1b, dinv, b1.reshape(1, dh), W2, s)

  acc2 = _edge_pass(npad, dc, cpw, gather=True, ntab=n)(
      g2, src32, dst32, jnp.zeros((_CHUNK, dc), jnp.float32))

  out = pl.pallas_call(
      _kc_body,
      grid=grid,
      in_specs=[
          pl.BlockSpec((2, bm, dc), lambda i: (0, i, 0)),
          pl.BlockSpec((bm, dc), lambda i: (i, 0)),
          pl.BlockSpec((bm, 1), lambda i: (i, 0)),
          pl.BlockSpec((1, dc), lambda i: (0, 0)),
      ],
      out_specs=pl.BlockSpec((bm, dc), lambda i: (i, 0)),
      out_shape=jax.ShapeDtypeStruct((n, dc), jnp.float32),
  )(acc2, g2, dinv, b2.reshape(1, dc))

  return out
